# Initial kernel scaffold; baseline (speedup 1.0000x reference)
import functools

import jax
import jax.numpy as jnp
from jax import lax
from jax.experimental import pallas as pl
from jax.experimental.pallas import tpu as pltpu

K = 8192
X = 256
Y = 256
N = X * Y
C2 = 192
ROWS_A = 32
ROWS_B = 16
INT_MIN = jnp.int32(-2**31)


def _sortable(g):
    bi = lax.bitcast_convert_type(g, jnp.int32)
    return jnp.where(bi >= 0, bi, INT_MIN - bi)


def _conv_kernel(in_ref, w_ref, g_ref):
    blk = in_ref[0]
    w0 = w_ref[0, :]
    w1 = w_ref[1, :]
    e = blk[:, 0, :, :]
    o = blk[:, 1, :, :]
    g = jnp.sum(e * w0[None, None, :], axis=-1)
    g = g + jnp.sum(o * w1[None, None, :], axis=-1)
    g_ref[0] = g


def _apply_kernel(g_ref, in_ref, out_ref, thr_ref, cut_ref):
    j = pl.program_id(1)

    @pl.when(j == 0)
    def _find_threshold():
        key = _sortable(g_ref[0])

        def t_body(k, t_u):
            bit = 31 - k
            cand_u = t_u | (jnp.int32(1) << bit)
            cand_s = cand_u ^ INT_MIN
            cnt = jnp.sum((key >= cand_s).astype(jnp.int32))
            return jnp.where(cnt >= K, cand_u, t_u)

        t_u = lax.fori_loop(0, 32, t_body, jnp.int32(0))
        t_s = t_u ^ INT_MIN
        cnt_gt = jnp.sum((key > t_s).astype(jnp.int32))
        r = K - cnt_gt
        tie = key == t_s
        rows = lax.broadcasted_iota(jnp.int32, (X, Y), 0)
        cols = lax.broadcasted_iota(jnp.int32, (X, Y), 1)
        flat = rows * Y + cols

        def c_body(k, c0):
            bit = 16 - k
            cand = c0 | (jnp.int32(1) << bit)
            cnt = jnp.sum((tie & (flat < cand)).astype(jnp.int32))
            return jnp.where(cnt < r, cand, c0)

        c0 = lax.fori_loop(0, 17, c_body, jnp.int32(0))
        thr_ref[0] = t_s
        cut_ref[0] = c0 + 1

    t_s = thr_ref[0]
    cut = cut_ref[0]
    g_blk = g_ref[0, pl.ds(j * ROWS_B, ROWS_B), :]
    key_blk = _sortable(g_blk)
    rows = lax.broadcasted_iota(jnp.int32, (ROWS_B, Y), 0) + j * ROWS_B
    cols = lax.broadcasted_iota(jnp.int32, (ROWS_B, Y), 1)
    flat = rows * Y + cols
    mask = (key_blk > t_s) | ((key_blk == t_s) & (flat < cut))
    mg = jnp.where(mask, g_blk, 0.0)
    out_ref[0] = in_ref[0] * mg[:, None, :, None]


@jax.jit
def kernel(inputs, gating_kernel):
    B = inputs.shape[0]
    in5 = inputs.reshape(B, X, 2, Y, C2)
    wr = gating_kernel[:, :, :, 0].reshape(2, C2)

    g = pl.pallas_call(
        _conv_kernel,
        grid=(B, X // ROWS_A),
        in_specs=[
            pl.BlockSpec((1, ROWS_A, 2, Y, C2), lambda b, j: (b, j, 0, 0, 0)),
            pl.BlockSpec((2, C2), lambda b, j: (0, 0)),
        ],
        out_specs=pl.BlockSpec((1, ROWS_A, Y), lambda b, j: (b, j, 0)),
        out_shape=jax.ShapeDtypeStruct((B, X, Y), jnp.float32),
    )(in5, wr)

    out5 = pl.pallas_call(
        _apply_kernel,
        grid=(B, X // ROWS_B),
        in_specs=[
            pl.BlockSpec((1, X, Y), lambda b, j: (b, 0, 0)),
            pl.BlockSpec((1, ROWS_B, 2, Y, C2), lambda b, j: (b, j, 0, 0, 0)),
        ],
        out_specs=pl.BlockSpec(
            (1, ROWS_B, 2, Y, C2), lambda b, j: (b, j, 0, 0, 0)),
        out_shape=jax.ShapeDtypeStruct((B, X, 2, Y, C2), jnp.float32),
        scratch_shapes=[
            pltpu.SMEM((1,), jnp.int32),
            pltpu.SMEM((1,), jnp.int32),
        ],
    )(g, in5)

    return out5.reshape(B, 2 * X, 2 * Y, 96)


# speedup vs baseline: 1.2942x; 1.2942x over previous
import functools

import jax
import numpy as np
import jax.numpy as jnp
from jax import lax
from jax.experimental import pallas as pl
from jax.experimental.pallas import tpu as pltpu

K = 8192
X = 256
Y = 256
N = X * Y
C2 = 192
ROWS_A = 32
ROWS_B = 16
INT_MIN = np.int32(-2**31)


def _sortable(g):
    bi = lax.bitcast_convert_type(g, jnp.int32)
    return jnp.where(bi >= 0, bi, INT_MIN - bi)


def _conv_kernel(in_ref, w_ref, g_ref):
    blk = in_ref[0]
    w0 = w_ref[0, :].astype(jnp.bfloat16).astype(jnp.float32)
    w1 = w_ref[1, :].astype(jnp.bfloat16).astype(jnp.float32)
    e = blk[:, 0, :, :].astype(jnp.bfloat16).astype(jnp.float32)
    o = blk[:, 1, :, :].astype(jnp.bfloat16).astype(jnp.float32)
    g = jnp.sum(e * w0[None, None, :], axis=-1)
    g = g + jnp.sum(o * w1[None, None, :], axis=-1)
    g_ref[0] = g


def _apply_kernel(g_ref, in_ref, out_ref, thr_ref, cut_ref):
    j = pl.program_id(1)

    @pl.when(j == 0)
    def _find_threshold():
        key = _sortable(g_ref[0])

        def t_body(k, t_u):
            bit = 31 - k
            cand_u = t_u | (np.int32(1) << bit)
            cand_s = cand_u ^ INT_MIN
            cnt = jnp.sum((key >= cand_s).astype(jnp.int32))
            return jnp.where(cnt >= K, cand_u, t_u)

        t_u = lax.fori_loop(0, 32, t_body, jnp.int32(0))
        t_s = t_u ^ INT_MIN
        cnt_gt = jnp.sum((key > t_s).astype(jnp.int32))
        r = K - cnt_gt
        tie = key == t_s
        rows = lax.broadcasted_iota(jnp.int32, (X, Y), 0)
        cols = lax.broadcasted_iota(jnp.int32, (X, Y), 1)
        flat = rows * Y + cols

        def c_body(k, c0):
            bit = 16 - k
            cand = c0 | (np.int32(1) << bit)
            cnt = jnp.sum((tie & (flat < cand)).astype(jnp.int32))
            return jnp.where(cnt < r, cand, c0)

        c0 = lax.fori_loop(0, 17, c_body, jnp.int32(0))
        thr_ref[0] = t_s
        cut_ref[0] = c0 + 1

    t_s = thr_ref[0]
    cut = cut_ref[0]
    g_blk = g_ref[0, pl.ds(j * ROWS_B, ROWS_B), :]
    key_blk = _sortable(g_blk)
    rows = lax.broadcasted_iota(jnp.int32, (ROWS_B, Y), 0) + j * ROWS_B
    cols = lax.broadcasted_iota(jnp.int32, (ROWS_B, Y), 1)
    flat = rows * Y + cols
    mask = (key_blk > t_s) | ((key_blk == t_s) & (flat < cut))
    mg = jnp.where(mask, g_blk, 0.0)
    out_ref[0] = in_ref[0] * mg[:, None, :, None]


@jax.jit
def kernel(inputs, gating_kernel):
    B = inputs.shape[0]
    in5 = inputs.reshape(B, X, 2, Y, C2)
    wr = gating_kernel[:, :, :, 0].reshape(2, C2)

    g = pl.pallas_call(
        _conv_kernel,
        grid=(B, X // ROWS_A),
        in_specs=[
            pl.BlockSpec((1, ROWS_A, 2, Y, C2), lambda b, j: (b, j, 0, 0, 0)),
            pl.BlockSpec((2, C2), lambda b, j: (0, 0)),
        ],
        out_specs=pl.BlockSpec((1, ROWS_A, Y), lambda b, j: (b, j, 0)),
        out_shape=jax.ShapeDtypeStruct((B, X, Y), jnp.float32),
    )(in5, wr)

    out5 = pl.pallas_call(
        _apply_kernel,
        grid=(B, X // ROWS_B),
        in_specs=[
            pl.BlockSpec((1, X, Y), lambda b, j: (b, 0, 0)),
            pl.BlockSpec((1, ROWS_B, 2, Y, C2), lambda b, j: (b, j, 0, 0, 0)),
        ],
        out_specs=pl.BlockSpec(
            (1, ROWS_B, 2, Y, C2), lambda b, j: (b, j, 0, 0, 0)),
        out_shape=jax.ShapeDtypeStruct((B, X, 2, Y, C2), jnp.float32),
        scratch_shapes=[
            pltpu.SMEM((1,), jnp.int32),
            pltpu.SMEM((1,), jnp.int32),
        ],
    )(g, in5)

    return out5.reshape(B, 2 * X, 2 * Y, 96)


# speedup vs baseline: 1.9681x; 1.5207x over previous
import jax
import numpy as np
import jax.numpy as jnp
from jax import lax
from jax.experimental import pallas as pl
from jax.experimental.pallas import tpu as pltpu

K = 8192
X = 256
Y = 256
C = 96
HROWS = 32
GR = HROWS // 2
INT_MIN = np.int32(-2**31)
HIGH = lax.Precision.HIGHEST


def _sortable(g):
    bi = lax.bitcast_convert_type(g, jnp.int32)
    return jnp.where(bi >= 0, bi, INT_MIN - bi)


def _conv_kernel(in_ref, wa_ref, wb_ref, se_ref, so_ref, tr_ref, g_ref):
    blk = in_ref[0].astype(jnp.bfloat16).astype(jnp.float32)
    wa = wa_ref[...].astype(jnp.bfloat16).astype(jnp.float32)
    wb = wb_ref[...].astype(jnp.bfloat16).astype(jnp.float32)
    pa = jnp.sum(blk * wa[:, None, :], axis=-1)
    pb = jnp.sum(blk * wb[:, None, :], axis=-1)
    g_rows = lax.dot_general(pa, se_ref[...], (((1,), (0,)), ((), ())),
                             precision=HIGH)
    g_rows += lax.dot_general(pb, so_ref[...], (((1,), (0,)), ((), ())),
                              precision=HIGH)
    g = lax.dot_general(tr_ref[...], g_rows, (((1,), (0,)), ((), ())),
                        precision=HIGH)
    g_ref[0] = g


def _apply_kernel(g_ref, in_ref, su_ref, tu_ref, out_ref, thr_ref, cut_ref):
    j = pl.program_id(1)

    @pl.when(j == 0)
    def _find_threshold():
        key = _sortable(g_ref[0])

        def t_body(k, t_u):
            bit = 31 - k
            cand_u = t_u | (np.int32(1) << bit)
            cand_s = cand_u ^ INT_MIN
            cnt = jnp.sum((key >= cand_s).astype(jnp.int32))
            return jnp.where(cnt >= K, cand_u, t_u)

        t_u = lax.fori_loop(0, 32, t_body, jnp.int32(0))
        t_s = t_u ^ INT_MIN
        cnt_gt = jnp.sum((key > t_s).astype(jnp.int32))
        r = K - cnt_gt
        tie = key == t_s
        rows = lax.broadcasted_iota(jnp.int32, (X, Y), 0)
        cols = lax.broadcasted_iota(jnp.int32, (X, Y), 1)
        flat = rows * Y + cols

        def c_body(k, c0):
            bit = 16 - k
            cand = c0 | (np.int32(1) << bit)
            cnt = jnp.sum((tie & (flat < cand)).astype(jnp.int32))
            return jnp.where(cnt < r, cand, c0)

        c0 = lax.fori_loop(0, 17, c_body, jnp.int32(0))
        thr_ref[0] = t_s
        cut_ref[0] = c0 + 1

    t_s = thr_ref[0]
    cut = cut_ref[0]
    g_blk = g_ref[0, pl.ds(j * GR, GR), :]
    key_blk = _sortable(g_blk)
    rows = lax.broadcasted_iota(jnp.int32, (GR, Y), 0) + j * GR
    cols = lax.broadcasted_iota(jnp.int32, (GR, Y), 1)
    flat = rows * Y + cols
    mask = (key_blk > t_s) | ((key_blk == t_s) & (flat < cut))
    mg = jnp.where(mask, g_blk, 0.0)
    u = lax.dot_general(mg, su_ref[...], (((1,), (0,)), ((), ())),
                        precision=HIGH)
    u = lax.dot_general(tu_ref[...], u, (((1,), (0,)), ((), ())),
                        precision=HIGH)
    out_ref[0] = in_ref[0] * u[:, :, None]


@jax.jit
def kernel(inputs, gating_kernel):
    B = inputs.shape[0]
    w = gating_kernel[:, :, :, 0]
    par = np.arange(HROWS) % 2
    wa = w[par, 0, :]
    wb = w[par, 1, :]
    yy = np.arange(Y)
    se = np.zeros((2 * Y, Y), np.float32)
    se[2 * yy, yy] = 1.0
    so = np.zeros((2 * Y, Y), np.float32)
    so[2 * yy + 1, yy] = 1.0
    tr = np.zeros((GR, HROWS), np.float32)
    tr[np.arange(GR), 2 * np.arange(GR)] = 1.0
    tr[np.arange(GR), 2 * np.arange(GR) + 1] = 1.0
    su = se.T.copy()
    su[yy, 2 * yy + 1] = 1.0
    tu = np.zeros((HROWS, GR), np.float32)
    tu[2 * np.arange(GR), np.arange(GR)] = 1.0
    tu[2 * np.arange(GR) + 1, np.arange(GR)] = 1.0

    g = pl.pallas_call(
        _conv_kernel,
        grid=(B, X // GR),
        in_specs=[
            pl.BlockSpec((1, HROWS, 2 * Y, C), lambda b, j: (b, j, 0, 0)),
            pl.BlockSpec((HROWS, C), lambda b, j: (0, 0)),
            pl.BlockSpec((HROWS, C), lambda b, j: (0, 0)),
            pl.BlockSpec((2 * Y, Y), lambda b, j: (0, 0)),
            pl.BlockSpec((2 * Y, Y), lambda b, j: (0, 0)),
            pl.BlockSpec((GR, HROWS), lambda b, j: (0, 0)),
        ],
        out_specs=pl.BlockSpec((1, GR, Y), lambda b, j: (b, j, 0)),
        out_shape=jax.ShapeDtypeStruct((B, X, Y), jnp.float32),
    )(inputs, wa, wb, se, so, tr)

    out = pl.pallas_call(
        _apply_kernel,
        grid=(B, X // GR),
        in_specs=[
            pl.BlockSpec((1, X, Y), lambda b, j: (b, 0, 0)),
            pl.BlockSpec((1, HROWS, 2 * Y, C), lambda b, j: (b, j, 0, 0)),
            pl.BlockSpec((Y, 2 * Y), lambda b, j: (0, 0)),
            pl.BlockSpec((HROWS, GR), lambda b, j: (0, 0)),
        ],
        out_specs=pl.BlockSpec(
            (1, HROWS, 2 * Y, C), lambda b, j: (b, j, 0, 0)),
        out_shape=jax.ShapeDtypeStruct((B, 2 * X, 2 * Y, C), jnp.float32),
        scratch_shapes=[
            pltpu.SMEM((1,), jnp.int32),
            pltpu.SMEM((1,), jnp.int32),
        ],
    )(g, inputs, su, tu)

    return out
